# Initial kernel scaffold; baseline (speedup 1.0000x reference)
import jax
import jax.numpy as jnp
from jax.experimental import pallas as pl


def kernel(node_feats, edge_attrs, tp_weights, sender_list, receiver_list):
    raise NotImplementedError("write your pallas kernel here")


# speedup vs baseline: 6.0249x; 6.0249x over previous
import functools

import jax
import jax.numpy as jnp
from jax import lax
from jax.experimental import pallas as pl
from jax.experimental.pallas import tpu as pltpu
from jax.experimental.pallas import tpu_sc as plsc

_LM_L = (0, 1, 1, 1, 2, 2, 2, 2, 2, 3, 3, 3, 3, 3, 3, 3)

_NC = 2
_NS = 16
_NW = _NC * _NS

_C = 48
_RB = 2000
_K = 16


def _sc_call(node_feats, edge_attrs, tp_weights, sender_list, receiver_list):
    N, F = node_feats.shape
    E = edge_attrs.shape[0]
    NLM = edge_attrs.shape[1]
    NCHUNK = -(-N // _C)
    ROUNDS = -(-NCHUNK // _NW)
    NLAST = N - (NCHUNK - 1) * _C
    NBLK = E // _RB
    assert NBLK * _RB == E and F % 16 == 0

    mesh = plsc.VectorSubcoreMesh(core_axis_name="c", subcore_axis_name="s")

    @functools.partial(
        pl.kernel,
        out_type=jax.ShapeDtypeStruct((N, NLM, F), jnp.float32),
        mesh=mesh,
        compiler_params=pltpu.CompilerParams(needs_layout_passes=False, use_tc_tiling_on_sc=False),
        scratch_types=[
            pltpu.VMEM((_C + 1, NLM, F), jnp.float32),
            pltpu.VMEM((_RB,), jnp.int32),
            pltpu.VMEM((_RB,), jnp.int32),
            pltpu.VMEM((_RB + 2 * _K,), jnp.int32),
            pltpu.VMEM((_RB + 2 * _K,), jnp.int32),
            pltpu.VMEM((_RB + 2 * _K,), jnp.int32),
            pltpu.VMEM((_K, 4, F), jnp.float32),
            pltpu.VMEM((_K, F), jnp.float32),
            pltpu.VMEM((_K, NLM), jnp.float32),
            pltpu.SemaphoreType.DMA,
            pltpu.SemaphoreType.DMA,
            pltpu.SemaphoreType.DMA,
        ],
    )
    def sc_kernel(nf_hbm, ea_hbm, tw_hbm, snd_hbm, rcv_hbm, out_hbm,
                  acc, rbuf, sbuf, meid, mrow, msnd, twv, nfv, eav,
                  sem0, sem1, sem2):
        wid = lax.axis_index("s") * _NC + lax.axis_index("c")
        iota = lax.iota(jnp.int32, 16)
        zeros16 = jnp.zeros((16,), jnp.float32)
        FV = F // 16

        def process_batches(nfull, _):
            def batch_body(b, _):
                bb = b * _K
                eidx = meid.at[pl.ds(bb, _K)]
                sidx = msnd.at[pl.ds(bb, _K)]
                cp_t = pltpu.async_copy(tw_hbm.at[eidx], twv, sem0)
                cp_e = pltpu.async_copy(ea_hbm.at[eidx], eav, sem1)
                cp_n = pltpu.async_copy(nf_hbm.at[sidx], nfv, sem2)
                cp_t.wait()
                cp_e.wait()
                cp_n.wait()

                def edge_body(k, _):
                    rl = mrow[pl.ds(bb + k, 16)][0]
                    kvec = jnp.full((16,), k, jnp.int32)
                    u = []
                    for j in range(FV):
                        sf = nfv[k, pl.ds(j * 16, 16)]
                        u.append([twv[k, l, pl.ds(j * 16, 16)] * sf
                                  for l in range(4)])
                    for lm in range(NLM):
                        ea_s = plsc.load_gather(
                            eav, [kvec, jnp.full((16,), lm, jnp.int32)])
                        l = _LM_L[lm]
                        for j in range(FV):
                            plsc.addupdate(
                                acc.at[rl, lm, pl.ds(j * 16, 16)],
                                ea_s * u[j][l])
                    return 0

                lax.fori_loop(0, _K, edge_body, 0)
                return 0

            lax.fori_loop(0, nfull, batch_body, 0)
            return 0

        def round_body(rnd, _):
            chunk = rnd * _NW + wid
            base = chunk * _C
            lo = base
            hi = base + _C

            def zero_body(i, _):
                for j in range(FV):
                    acc[i >> 4, i & 15, pl.ds(j * 16, 16)] = zeros16
                return 0
            lax.fori_loop(0, (_C + 1) * NLM, zero_body, 0)

            def block_body(blk, cursor):
                ebase = blk * _RB
                pltpu.sync_copy(rcv_hbm.at[pl.ds(ebase, _RB)], rbuf)
                pltpu.sync_copy(snd_hbm.at[pl.ds(ebase, _RB)], sbuf)

                def scan_body(i, cursor):
                    r = rbuf[pl.ds(i * 16, 16)]
                    m = (r >= lo) & (r < hi)
                    pc = jnp.sum(m.astype(jnp.int32), axis=0)

                    @pl.when(pc > 0)
                    def _():
                        e_vec = (ebase + i * 16) + iota
                        plsc.store_compressed(
                            meid.at[pl.ds(cursor, 16)], e_vec, mask=m)
                        plsc.store_compressed(
                            mrow.at[pl.ds(cursor, 16)], r - lo, mask=m)
                        plsc.store_compressed(
                            msnd.at[pl.ds(cursor, 16)],
                            sbuf[pl.ds(i * 16, 16)], mask=m)
                    return cursor + pc

                cursor = lax.fori_loop(0, _RB // 16, scan_body, cursor)
                nfull = cursor >> 4
                process_batches(nfull, None)
                tail = cursor & ~15

                @pl.when(nfull > 0)
                def _():
                    e_t = meid[pl.ds(tail, 16)]
                    r_t = mrow[pl.ds(tail, 16)]
                    s_t = msnd[pl.ds(tail, 16)]
                    meid[pl.ds(0, 16)] = e_t
                    mrow[pl.ds(0, 16)] = r_t
                    msnd[pl.ds(0, 16)] = s_t
                return cursor & 15

            cursor = lax.fori_loop(0, NBLK, block_body, jnp.int32(0))

            @pl.when(cursor > 0)
            def _():
                meid[pl.ds(cursor, 16)] = jnp.zeros((16,), jnp.int32)
                mrow[pl.ds(cursor, 16)] = jnp.full((16,), _C, jnp.int32)
                msnd[pl.ds(cursor, 16)] = jnp.zeros((16,), jnp.int32)
                process_batches(jnp.int32(1), None)

            @pl.when(chunk < NCHUNK - 1)
            def _():
                pltpu.sync_copy(acc.at[pl.ds(0, _C)],
                                out_hbm.at[pl.ds(base, _C)])

            @pl.when(chunk == NCHUNK - 1)
            def _():
                pltpu.sync_copy(acc.at[pl.ds(0, NLAST)],
                                out_hbm.at[pl.ds(base, NLAST)])
            return 0

        lax.fori_loop(0, ROUNDS, round_body, 0)

    return sc_kernel(node_feats, edge_attrs, tp_weights, sender_list,
                     receiver_list)


def kernel(node_feats, edge_attrs, tp_weights, sender_list, receiver_list):
    return _sc_call(node_feats, edge_attrs, tp_weights, sender_list,
                    receiver_list)


# speedup vs baseline: 7.9011x; 1.3114x over previous
import functools

import jax
import jax.numpy as jnp
from jax import lax
from jax.experimental import pallas as pl
from jax.experimental.pallas import tpu as pltpu
from jax.experimental.pallas import tpu_sc as plsc

_LM_L = (0, 1, 1, 1, 2, 2, 2, 2, 2, 3, 3, 3, 3, 3, 3, 3)

_NC = 2
_NS = 16
_NW = _NC * _NS

_C = 47
_RB = 800
_K = 16


def _sc_call(node_feats, edge_attrs, tp_weights, sender_list, receiver_list):
    N, F = node_feats.shape
    E = edge_attrs.shape[0]
    NLM = edge_attrs.shape[1]
    NCHUNK = -(-N // _C)
    ROUNDS = -(-NCHUNK // _NW)
    NLAST = N - (NCHUNK - 1) * _C
    NBLK = E // _RB
    assert NBLK * _RB == E and F % 16 == 0

    mesh = plsc.VectorSubcoreMesh(core_axis_name="c", subcore_axis_name="s")

    @functools.partial(
        pl.kernel,
        out_type=jax.ShapeDtypeStruct((N, NLM, F), jnp.float32),
        mesh=mesh,
        compiler_params=pltpu.CompilerParams(
            needs_layout_passes=False, use_tc_tiling_on_sc=False),
        scratch_types=[
            pltpu.VMEM((_C + 1, NLM, F), jnp.float32),
            pltpu.VMEM((2, _RB), jnp.int32),
            pltpu.VMEM((2, _RB), jnp.int32),
            pltpu.VMEM((_RB + 2 * _K,), jnp.int32),
            pltpu.VMEM((_RB + 2 * _K,), jnp.int32),
            pltpu.VMEM((_RB + 2 * _K,), jnp.int32),
            pltpu.VMEM((2, _K, 4, F), jnp.float32),
            pltpu.VMEM((2, _K, F), jnp.float32),
            pltpu.VMEM((2, _K, NLM), jnp.float32),
            pltpu.SemaphoreType.DMA,
            pltpu.SemaphoreType.DMA,
            pltpu.SemaphoreType.DMA,
            pltpu.SemaphoreType.DMA,
            pltpu.SemaphoreType.DMA,
        ],
    )
    def sc_kernel(nf_hbm, ea_hbm, tw_hbm, snd_hbm, rcv_hbm, out_hbm,
                  acc, rbuf, sbuf, meid, mrow, msnd, twv, nfv, eav,
                  semr, sems, semt, seme, semn):
        wid = lax.axis_index("s") * _NC + lax.axis_index("c")
        iota = lax.iota(jnp.int32, 16)
        zeros16 = jnp.zeros((16,), jnp.float32)
        FV = F // 16

        def fire_block(blk, slot):
            pltpu.async_copy(rcv_hbm.at[pl.ds(blk * _RB, _RB)],
                             rbuf.at[slot], semr)
            pltpu.async_copy(snd_hbm.at[pl.ds(blk * _RB, _RB)],
                             sbuf.at[slot], sems)

        def wait_block(slot):
            pltpu.make_async_copy(rcv_hbm.at[pl.ds(0, _RB)],
                                  rbuf.at[slot], semr).wait()
            pltpu.make_async_copy(snd_hbm.at[pl.ds(0, _RB)],
                                  sbuf.at[slot], sems).wait()

        def fire_batch(b, slot):
            bb = b * _K
            eidx = meid.at[pl.ds(bb, _K)]
            sidx = msnd.at[pl.ds(bb, _K)]
            pltpu.async_copy(tw_hbm.at[eidx], twv.at[slot], semt)
            pltpu.async_copy(ea_hbm.at[eidx], eav.at[slot], seme)
            pltpu.async_copy(nf_hbm.at[sidx], nfv.at[slot], semn)

        def wait_batch(slot):
            idx0 = meid.at[pl.ds(0, _K)]
            pltpu.make_async_copy(tw_hbm.at[idx0], twv.at[slot], semt).wait()
            pltpu.make_async_copy(ea_hbm.at[idx0], eav.at[slot], seme).wait()
            pltpu.make_async_copy(nf_hbm.at[idx0], nfv.at[slot], semn).wait()

        def compute_batch(b, slot):
            bb = b * _K

            def edge_body(k, _):
                rl = mrow[pl.ds(bb + k, 16)][0]
                kvec = jnp.full((16,), k, jnp.int32)
                u = []
                for j in range(FV):
                    sf = nfv[slot, k, pl.ds(j * 16, 16)]
                    u.append([twv[slot, k, l, pl.ds(j * 16, 16)] * sf
                              for l in range(4)])
                for lm in range(NLM):
                    ea_s = plsc.load_gather(
                        eav.at[slot], [kvec, jnp.full((16,), lm, jnp.int32)])
                    l = _LM_L[lm]
                    for j in range(FV):
                        plsc.addupdate(
                            acc.at[rl, lm, pl.ds(j * 16, 16)],
                            ea_s * u[j][l])
                return 0

            lax.fori_loop(0, _K, edge_body, 0)

        def process_batches(nfull):
            @pl.when(nfull > 0)
            def _():
                fire_batch(0, 0)

                def batch_body(b, _):
                    bslot = b & 1
                    wait_batch(bslot)

                    @pl.when(b + 1 < nfull)
                    def _():
                        fire_batch(b + 1, 1 - bslot)
                    compute_batch(b, bslot)
                    return 0

                lax.fori_loop(0, nfull, batch_body, 0)

        def round_body(rnd, _):
            chunk = rnd * _NW + wid
            base = chunk * _C
            lo = base

            def zero_body(i, _):
                for j in range(FV):
                    acc[i >> 4, i & 15, pl.ds(j * 16, 16)] = zeros16
                return 0
            lax.fori_loop(0, (_C + 1) * NLM, zero_body, 0)

            fire_block(0, 0)

            def block_body(blk, cursor):
                slot = blk & 1
                wait_block(slot)

                @pl.when(blk + 1 < NBLK)
                def _():
                    fire_block(blk + 1, 1 - slot)

                def scan_body(i, cursor):
                    r = rbuf[slot, pl.ds(i * 16, 16)]
                    rl_vec = r - lo
                    m = plsc.bitcast(rl_vec, jnp.uint32) < jnp.uint32(_C)
                    pc = plsc.all_reduce_population_count(m)[0]

                    @pl.when(pc > 0)
                    def _():
                        e_vec = (blk * _RB + i * 16) + iota
                        plsc.store_compressed(
                            meid.at[pl.ds(cursor, 16)], e_vec, mask=m)
                        plsc.store_compressed(
                            mrow.at[pl.ds(cursor, 16)], rl_vec, mask=m)
                        plsc.store_compressed(
                            msnd.at[pl.ds(cursor, 16)],
                            sbuf[slot, pl.ds(i * 16, 16)], mask=m)
                    return cursor + pc

                cursor = lax.fori_loop(0, _RB // 16, scan_body, cursor)
                nfull = cursor >> 4
                process_batches(nfull)
                tail = cursor & ~15

                @pl.when(nfull > 0)
                def _():
                    e_t = meid[pl.ds(tail, 16)]
                    r_t = mrow[pl.ds(tail, 16)]
                    s_t = msnd[pl.ds(tail, 16)]
                    meid[pl.ds(0, 16)] = e_t
                    mrow[pl.ds(0, 16)] = r_t
                    msnd[pl.ds(0, 16)] = s_t
                return cursor & 15

            cursor = lax.fori_loop(0, NBLK, block_body, jnp.int32(0))

            @pl.when(cursor > 0)
            def _():
                meid[pl.ds(cursor, 16)] = jnp.zeros((16,), jnp.int32)
                mrow[pl.ds(cursor, 16)] = jnp.full((16,), _C, jnp.int32)
                msnd[pl.ds(cursor, 16)] = jnp.zeros((16,), jnp.int32)
                fire_batch(0, 0)
                wait_batch(0)
                compute_batch(0, 0)

            @pl.when(chunk < NCHUNK - 1)
            def _():
                pltpu.sync_copy(acc.at[pl.ds(0, _C)],
                                out_hbm.at[pl.ds(base, _C)])

            @pl.when(chunk == NCHUNK - 1)
            def _():
                pltpu.sync_copy(acc.at[pl.ds(0, NLAST)],
                                out_hbm.at[pl.ds(base, NLAST)])
            return 0

        lax.fori_loop(0, ROUNDS, round_body, 0)

    return sc_kernel(node_feats, edge_attrs, tp_weights, sender_list,
                     receiver_list)


def kernel(node_feats, edge_attrs, tp_weights, sender_list, receiver_list):
    return _sc_call(node_feats, edge_attrs, tp_weights, sender_list,
                    receiver_list)
